# baseline (speedup 1.0000x reference)
import functools

import jax
import jax.numpy as jnp
from jax import lax
from jax.experimental import pallas as pl
from jax.experimental.pallas import tpu as pltpu
from jax.experimental.pallas import tpu_sc as plsc

B = 8
N = 512
D = 32
BINS = 128
M = N * N
EPS = 1e-8

NC = 2
NS = 16
CHUNK = M // NS


def _dist_body(x1_ref, x2_ref, dist_ref, max_ref):
    ones_row = jnp.ones((1, D), jnp.float32)

    def dmat(x):
        g = lax.dot_general(x, x, (((1,), (1,)), ((), ())),
                            preferred_element_type=jnp.float32,
                            precision=lax.Precision.HIGHEST)
        xsq = x * x
        ncol = lax.dot_general(xsq, ones_row, (((1,), (1,)), ((), ())),
                               preferred_element_type=jnp.float32,
                               precision=lax.Precision.HIGHEST)
        nrow = lax.dot_general(ones_row, xsq, (((1,), (1,)), ((), ())),
                               preferred_element_type=jnp.float32,
                               precision=lax.Precision.HIGHEST)
        s = ncol + nrow - 2.0 * g
        return jnp.sqrt(jnp.maximum(s, 0.0))

    d1 = dmat(x1_ref[0])
    d2 = dmat(x2_ref[0])
    dist_ref[0, 0] = d1
    dist_ref[1, 0] = d2
    max_ref[0, 0, 0] = jnp.maximum(jnp.max(d1), jnp.max(d2))


def _hist_body(dists_hbm, maxs_hbm, out_hbm, maxs_v, buf, hist, pub,
               sem0, sem1):
    c = lax.axis_index("c")
    s = lax.axis_index("s")
    pltpu.sync_copy(maxs_hbm, maxs_v)
    lane = lax.iota(jnp.int32, NS)
    i16v = lane * 16
    ones_v = jnp.ones((16,), jnp.float32)
    zero_v = jnp.zeros((16,), jnp.float32)
    sems = (sem0, sem1)

    scale_vec = (BINS * 1.0) / maxs_v[...]

    copies = [pltpu.async_copy(dists_hbm.at[c, 0, s], buf.at[0], sems[0])]

    @plsc.parallel_loop(0, (B * BINS * NS) // 64)
    def zero_body(i):
        base = i * 64
        for k in range(4):
            hist[pl.ds(base + k * 16, 16)] = zero_v

    for b in range(B):
        copies[b].wait()
        if b + 1 < B:
            copies.append(pltpu.async_copy(
                dists_hbm.at[c, b + 1, s], buf.at[(b + 1) & 1],
                sems[(b + 1) & 1]))
        scale = scale_vec[b]
        base_v = lane + (b * BINS * NS)
        cur = b & 1

        @plsc.parallel_loop(0, CHUNK // 128, unroll=4)
        def bin_body(i):
            base = i * 128
            for k in range(8):
                x = buf[cur, pl.ds(base + k * 16, 16)]
                bi = jnp.minimum((x * scale).astype(jnp.int32), BINS - 1)
                addr = base_v + lax.shift_left(bi, 4)
                plsc.addupdate_scatter(hist, [addr], ones_v)

    for b in range(B):
        @plsc.parallel_loop(0, BINS // 16)
        def red_body(g):
            gbase = b * (BINS * NS) + g * 256
            acc = plsc.load_gather(hist, [i16v + gbase])
            for l in range(1, NS):
                acc = acc + plsc.load_gather(hist, [i16v + (gbase + l)])
            pub[b, pl.ds(g * 16, 16)] = acc

    pltpu.sync_copy(pub, out_hbm.at[c, s])


def _jsd_body(counts_ref, maxs_ref, out_ref):
    cts = counts_ref[...]
    mxv = maxs_ref[...]
    summed = jnp.sum(cts, axis=1)
    c1 = summed[0]
    c2 = summed[1]
    j = lax.broadcasted_iota(jnp.int32, (B, BINS), 1).astype(jnp.float32)
    w = mxv * ((j + 1.0) * (1.0 / BINS)) - mxv * (j * (1.0 / BINS))
    mw = float(M) * w
    px = c1 / mw
    qx = c2 / mw
    pm = (px + qx) * 0.5
    lpm = jnp.log(pm + EPS)
    e1 = jnp.sum(px * (jnp.log(px + EPS) - lpm), axis=1, keepdims=True)
    e2 = jnp.sum(qx * (jnp.log(qx + EPS) - lpm), axis=1, keepdims=True)
    out_ref[...] = (e1 + e2) * 0.5


def _make_hist_kernel():
    mesh = plsc.VectorSubcoreMesh(core_axis_name="c", subcore_axis_name="s")
    return pl.kernel(
        _hist_body,
        out_type=jax.ShapeDtypeStruct((NC, NS, B, BINS), jnp.float32),
        mesh=mesh,
        compiler_params=pltpu.CompilerParams(needs_layout_passes=False),
        scratch_types=[
            pltpu.VMEM((16,), jnp.float32),
            pltpu.VMEM((2, CHUNK), jnp.float32),
            pltpu.VMEM((B * BINS * NS,), jnp.float32),
            pltpu.VMEM((B, BINS), jnp.float32),
            pltpu.SemaphoreType.DMA,
            pltpu.SemaphoreType.DMA,
        ],
    )


def kernel(data1, data2):
    dists, maxs = pl.pallas_call(
        _dist_body,
        grid=(B,),
        in_specs=[
            pl.BlockSpec((1, N, D), lambda b: (b, 0, 0)),
            pl.BlockSpec((1, N, D), lambda b: (b, 0, 0)),
        ],
        out_specs=[
            pl.BlockSpec((2, 1, N, N), lambda b: (0, b, 0, 0)),
            pl.BlockSpec((1, 1, 1), lambda b: (b, 0, 0),
                         memory_space=pltpu.SMEM),
        ],
        out_shape=[
            jax.ShapeDtypeStruct((2, B, N, N), jnp.float32),
            jax.ShapeDtypeStruct((B, 1, 1), jnp.float32),
        ],
    )(data1, data2)

    dists_r = dists.reshape(2, B, NS, CHUNK)
    maxs_pad = jnp.concatenate(
        [maxs.reshape(B), jnp.ones((16 - B,), jnp.float32)])

    counts = _make_hist_kernel()(dists_r, maxs_pad)

    jsd = pl.pallas_call(
        _jsd_body,
        in_specs=[
            pl.BlockSpec((NC, NS, B, BINS), lambda: (0, 0, 0, 0)),
            pl.BlockSpec((B, 1), lambda: (0, 0)),
        ],
        out_specs=pl.BlockSpec((B, 1), lambda: (0, 0)),
        out_shape=jax.ShapeDtypeStruct((B, 1), jnp.float32),
    )(counts, maxs.reshape(B, 1))
    return jsd.reshape(B)


# speedup vs baseline: 1.0527x; 1.0527x over previous
import functools

import jax
import jax.numpy as jnp
from jax import lax
from jax.experimental import pallas as pl
from jax.experimental.pallas import tpu as pltpu
from jax.experimental.pallas import tpu_sc as plsc

B = 8
N = 512
D = 32
BINS = 128
M = N * N
EPS = 1e-8

NC = 2
NS = 16
CHUNK = M // NS


def _dist_body(x1_ref, x2_ref, dist_ref, max_ref):
    ones_row = jnp.ones((1, D), jnp.float32)

    def dmat(x):
        xh = x.astype(jnp.bfloat16)
        xl = (x - xh.astype(jnp.float32)).astype(jnp.bfloat16)
        dims = (((1,), (1,)), ((), ()))
        g = (lax.dot_general(xh, xh, dims, preferred_element_type=jnp.float32)
             + lax.dot_general(xh, xl, dims, preferred_element_type=jnp.float32)
             + lax.dot_general(xl, xh, dims, preferred_element_type=jnp.float32))
        xsq = x * x
        ncol = lax.dot_general(xsq, ones_row, (((1,), (1,)), ((), ())),
                               preferred_element_type=jnp.float32,
                               precision=lax.Precision.HIGHEST)
        nrow = lax.dot_general(ones_row, xsq, (((1,), (1,)), ((), ())),
                               preferred_element_type=jnp.float32,
                               precision=lax.Precision.HIGHEST)
        s = ncol + nrow - 2.0 * g
        return jnp.sqrt(jnp.maximum(s, 0.0))

    d1 = dmat(x1_ref[0])
    d2 = dmat(x2_ref[0])
    dist_ref[0, 0] = d1
    dist_ref[1, 0] = d2
    max_ref[0, 0, 0] = jnp.maximum(jnp.max(d1), jnp.max(d2))


def _hist_body(dists_hbm, maxs_hbm, out_hbm, maxs_v, buf, hist, pub,
               sem0, sem1):
    c = lax.axis_index("c")
    s = lax.axis_index("s")
    pltpu.sync_copy(maxs_hbm, maxs_v)
    lane = lax.iota(jnp.int32, NS)
    i16v = lane * 16
    ones_v = jnp.ones((16,), jnp.float32)
    zero_v = jnp.zeros((16,), jnp.float32)
    sems = (sem0, sem1)

    scale_vec = (BINS * 1.0) / maxs_v[...]

    copies = [pltpu.async_copy(dists_hbm.at[c, 0, s], buf.at[0], sems[0])]

    @plsc.parallel_loop(0, (B * BINS * NS) // 64)
    def zero_body(i):
        base = i * 64
        for k in range(4):
            hist[pl.ds(base + k * 16, 16)] = zero_v

    for b in range(B):
        copies[b].wait()
        if b + 1 < B:
            copies.append(pltpu.async_copy(
                dists_hbm.at[c, b + 1, s], buf.at[(b + 1) & 1],
                sems[(b + 1) & 1]))
        scale = scale_vec[b]
        base_v = lane + (b * BINS * NS)
        cur = b & 1

        @plsc.parallel_loop(0, CHUNK // 128, unroll=4)
        def bin_body(i):
            base = i * 128
            for k in range(8):
                x = buf[cur, pl.ds(base + k * 16, 16)]
                bi = jnp.minimum((x * scale).astype(jnp.int32), BINS - 1)
                addr = base_v + lax.shift_left(bi, 4)
                plsc.addupdate_scatter(hist, [addr], ones_v)

    for b in range(B):
        @plsc.parallel_loop(0, BINS // 16)
        def red_body(g):
            gbase = b * (BINS * NS) + g * 256
            acc = plsc.load_gather(hist, [i16v + gbase])
            for l in range(1, NS):
                acc = acc + plsc.load_gather(hist, [i16v + (gbase + l)])
            pub[b, pl.ds(g * 16, 16)] = acc

    pltpu.sync_copy(pub, out_hbm.at[c, s])


def _jsd_body(counts_ref, maxs_ref, out_ref):
    cts = counts_ref[...]
    mxv = maxs_ref[...]
    summed = jnp.sum(cts, axis=1)
    c1 = summed[0]
    c2 = summed[1]
    j = lax.broadcasted_iota(jnp.int32, (B, BINS), 1).astype(jnp.float32)
    w = mxv * ((j + 1.0) * (1.0 / BINS)) - mxv * (j * (1.0 / BINS))
    mw = float(M) * w
    px = c1 / mw
    qx = c2 / mw
    pm = (px + qx) * 0.5
    lpm = jnp.log(pm + EPS)
    e1 = jnp.sum(px * (jnp.log(px + EPS) - lpm), axis=1, keepdims=True)
    e2 = jnp.sum(qx * (jnp.log(qx + EPS) - lpm), axis=1, keepdims=True)
    out_ref[...] = (e1 + e2) * 0.5


def _make_hist_kernel():
    mesh = plsc.VectorSubcoreMesh(core_axis_name="c", subcore_axis_name="s")
    return pl.kernel(
        _hist_body,
        out_type=jax.ShapeDtypeStruct((NC, NS, B, BINS), jnp.float32),
        mesh=mesh,
        compiler_params=pltpu.CompilerParams(needs_layout_passes=False),
        scratch_types=[
            pltpu.VMEM((16,), jnp.float32),
            pltpu.VMEM((2, CHUNK), jnp.float32),
            pltpu.VMEM((B * BINS * NS,), jnp.float32),
            pltpu.VMEM((B, BINS), jnp.float32),
            pltpu.SemaphoreType.DMA,
            pltpu.SemaphoreType.DMA,
        ],
    )


def kernel(data1, data2):
    dists, maxs = pl.pallas_call(
        _dist_body,
        grid=(B,),
        in_specs=[
            pl.BlockSpec((1, N, D), lambda b: (b, 0, 0)),
            pl.BlockSpec((1, N, D), lambda b: (b, 0, 0)),
        ],
        out_specs=[
            pl.BlockSpec((2, 1, N, N), lambda b: (0, b, 0, 0)),
            pl.BlockSpec((1, 1, 1), lambda b: (b, 0, 0),
                         memory_space=pltpu.SMEM),
        ],
        out_shape=[
            jax.ShapeDtypeStruct((2, B, N, N), jnp.float32),
            jax.ShapeDtypeStruct((B, 1, 1), jnp.float32),
        ],
    )(data1, data2)

    dists_r = dists.reshape(2, B, NS, CHUNK)
    maxs_pad = jnp.concatenate(
        [maxs.reshape(B), jnp.ones((16 - B,), jnp.float32)])

    counts = _make_hist_kernel()(dists_r, maxs_pad)

    jsd = pl.pallas_call(
        _jsd_body,
        in_specs=[
            pl.BlockSpec((NC, NS, B, BINS), lambda: (0, 0, 0, 0)),
            pl.BlockSpec((B, 1), lambda: (0, 0)),
        ],
        out_specs=pl.BlockSpec((B, 1), lambda: (0, 0)),
        out_shape=jax.ShapeDtypeStruct((B, 1), jnp.float32),
    )(counts, maxs.reshape(B, 1))
    return jsd.reshape(B)


# speedup vs baseline: 1.0768x; 1.0229x over previous
import functools

import jax
import jax.numpy as jnp
from jax import lax
from jax.experimental import pallas as pl
from jax.experimental.pallas import tpu as pltpu
from jax.experimental.pallas import tpu_sc as plsc

B = 8
N = 512
D = 32
BINS = 128
M = N * N
EPS = 1e-8

NC = 2
NS = 16
CHUNK = M // NS
STRIDE = (BINS + 1) * NS


def _dist_body(x1_ref, x2_ref, dist_ref, max_ref):
    ones_row = jnp.ones((1, D), jnp.float32)

    def dmat(x):
        xh = x.astype(jnp.bfloat16)
        xl = (x - xh.astype(jnp.float32)).astype(jnp.bfloat16)
        dims = (((1,), (1,)), ((), ()))
        g = (lax.dot_general(xh, xh, dims, preferred_element_type=jnp.float32)
             + lax.dot_general(xh, xl, dims, preferred_element_type=jnp.float32)
             + lax.dot_general(xl, xh, dims, preferred_element_type=jnp.float32))
        xsq = x * x
        ncol = lax.dot_general(xsq, ones_row, (((1,), (1,)), ((), ())),
                               preferred_element_type=jnp.float32,
                               precision=lax.Precision.HIGHEST)
        nrow = lax.dot_general(ones_row, xsq, (((1,), (1,)), ((), ())),
                               preferred_element_type=jnp.float32,
                               precision=lax.Precision.HIGHEST)
        s = ncol + nrow - 2.0 * g
        return jnp.sqrt(jnp.maximum(s, 0.0))

    d1 = dmat(x1_ref[0])
    d2 = dmat(x2_ref[0])
    dist_ref[0, 0] = d1
    dist_ref[1, 0] = d2
    max_ref[0, 0, 0] = jnp.maximum(jnp.max(d1), jnp.max(d2))


def _hist_body(dists_hbm, maxs_hbm, out_hbm, maxs_v, buf, hist, pub,
               sem0, sem1):
    c = lax.axis_index("c")
    s = lax.axis_index("s")
    pltpu.sync_copy(maxs_hbm, maxs_v)
    lane = lax.iota(jnp.int32, NS)
    i16v = lane * 16
    ones_v = jnp.ones((16,), jnp.float32)
    zero_v = jnp.zeros((16,), jnp.float32)
    sems = (sem0, sem1)

    scale_vec = (BINS * 16.0) / maxs_v[...]

    copies = [pltpu.async_copy(dists_hbm.at[c, 0, s], buf.at[0], sems[0])]

    @plsc.parallel_loop(0, (B * STRIDE) // 64)
    def zero_body(i):
        base = i * 64
        for k in range(4):
            hist[pl.ds(base + k * 16, 16)] = zero_v

    for b in range(B):
        copies[b].wait()
        if b + 1 < B:
            copies.append(pltpu.async_copy(
                dists_hbm.at[c, b + 1, s], buf.at[(b + 1) & 1],
                sems[(b + 1) & 1]))
        scale = scale_vec[b]
        base_v = lane + (b * STRIDE)
        cur = b & 1

        @plsc.parallel_loop(0, CHUNK // 128, unroll=8)
        def bin_body(i):
            base = i * 128
            for k in range(8):
                x = buf[cur, pl.ds(base + k * 16, 16)]
                v = (x * scale).astype(jnp.int32)
                addr = base_v + jnp.bitwise_and(v, -16)
                plsc.addupdate_scatter(hist, [addr], ones_v)

    for b in range(B):
        @plsc.parallel_loop(0, BINS // 16 - 1)
        def red_body(g):
            gbase = b * STRIDE + g * 256
            acc = plsc.load_gather(hist, [i16v + gbase])
            for l in range(1, NS):
                acc = acc + plsc.load_gather(hist, [i16v + (gbase + l)])
            pub[b, pl.ds(g * 16, 16)] = acc

        gbase = b * STRIDE + (BINS // 16 - 1) * 256
        acc = plsc.load_gather(hist, [i16v + gbase])
        for l in range(1, NS):
            acc = acc + plsc.load_gather(hist, [i16v + (gbase + l)])
        ov = hist[pl.ds(b * STRIDE + BINS * NS, 16)]
        acc = acc + jnp.where(lane == NS - 1, jnp.sum(ov), 0.0)
        pub[b, pl.ds(BINS - 16, 16)] = acc

    pltpu.sync_copy(pub, out_hbm.at[c, s])


def _jsd_body(counts_ref, maxs_ref, out_ref):
    cts = counts_ref[...]
    mxv = maxs_ref[...][0:B]
    summed = jnp.sum(cts, axis=1)
    c1 = summed[0]
    c2 = summed[1]
    j = lax.broadcasted_iota(jnp.int32, (B, BINS), 1).astype(jnp.float32)
    w = mxv * ((j + 1.0) * (1.0 / BINS)) - mxv * (j * (1.0 / BINS))
    mw = float(M) * w
    px = c1 / mw
    qx = c2 / mw
    pm = (px + qx) * 0.5
    lpm = jnp.log(pm + EPS)
    e1 = jnp.sum(px * (jnp.log(px + EPS) - lpm), axis=1, keepdims=True)
    e2 = jnp.sum(qx * (jnp.log(qx + EPS) - lpm), axis=1, keepdims=True)
    out_ref[...] = (e1 + e2) * 0.5


def _make_hist_kernel():
    mesh = plsc.VectorSubcoreMesh(core_axis_name="c", subcore_axis_name="s")
    return pl.kernel(
        _hist_body,
        out_type=jax.ShapeDtypeStruct((NC, NS, B, BINS), jnp.float32),
        mesh=mesh,
        compiler_params=pltpu.CompilerParams(needs_layout_passes=False),
        scratch_types=[
            pltpu.VMEM((16,), jnp.float32),
            pltpu.VMEM((2, CHUNK), jnp.float32),
            pltpu.VMEM((B * STRIDE,), jnp.float32),
            pltpu.VMEM((B, BINS), jnp.float32),
            pltpu.SemaphoreType.DMA,
            pltpu.SemaphoreType.DMA,
        ],
    )


def kernel(data1, data2):
    dists, maxs = pl.pallas_call(
        _dist_body,
        grid=(B,),
        in_specs=[
            pl.BlockSpec((1, N, D), lambda b: (b, 0, 0)),
            pl.BlockSpec((1, N, D), lambda b: (b, 0, 0)),
        ],
        out_specs=[
            pl.BlockSpec((2, 1, N, N), lambda b: (0, b, 0, 0)),
            pl.BlockSpec((1, 1, 1), lambda b: (b, 0, 0),
                         memory_space=pltpu.SMEM),
        ],
        out_shape=[
            jax.ShapeDtypeStruct((2, B, N, N), jnp.float32),
            jax.ShapeDtypeStruct((16, 1, 1), jnp.float32),
        ],
    )(data1, data2)

    dists_r = dists.reshape(2, B, NS, CHUNK)

    counts = _make_hist_kernel()(dists_r, maxs.reshape(16))

    jsd = pl.pallas_call(
        _jsd_body,
        in_specs=[
            pl.BlockSpec((NC, NS, B, BINS), lambda: (0, 0, 0, 0)),
            pl.BlockSpec((16, 1), lambda: (0, 0)),
        ],
        out_specs=pl.BlockSpec((B, 1), lambda: (0, 0)),
        out_shape=jax.ShapeDtypeStruct((B, 1), jnp.float32),
    )(counts, maxs.reshape(16, 1))
    return jsd.reshape(B)


# speedup vs baseline: 1.4743x; 1.3691x over previous
import functools

import jax
import jax.numpy as jnp
from jax import lax
from jax.experimental import pallas as pl
from jax.experimental.pallas import tpu as pltpu
from jax.experimental.pallas import tpu_sc as plsc

B = 8
N = 512
H = N // 2
D = 32
BINS = 128
M = N * N
MH = M // 2
EPS = 1e-8

NC = 2
NS = 16
CHUNK = MH // NS
STRIDE = (BINS + 1) * NS


def _dist_body(x1_ref, x2_ref, dist_ref, max_ref):
    ones_row = jnp.ones((1, D), jnp.float32)
    ii = lax.broadcasted_iota(jnp.int32, (H, H), 0)
    jj = lax.broadcasted_iota(jnp.int32, (H, H), 1)

    def dmat(x):
        xh = x.astype(jnp.bfloat16)
        xl = (x - xh.astype(jnp.float32)).astype(jnp.bfloat16)
        dims = (((1,), (1,)), ((), ()))

        def dot3(a_h, a_l, b_h, b_l):
            return (lax.dot_general(a_h, b_h, dims,
                                    preferred_element_type=jnp.float32)
                    + lax.dot_general(a_h, b_l, dims,
                                      preferred_element_type=jnp.float32)
                    + lax.dot_general(a_l, b_h, dims,
                                      preferred_element_type=jnp.float32))

        g_top = dot3(xh[0:H], xl[0:H], xh, xl)
        g_br = dot3(xh[H:N], xl[H:N], xh[H:N], xl[H:N])
        xsq = x * x
        ncol = lax.dot_general(xsq, ones_row, (((1,), (1,)), ((), ())),
                               preferred_element_type=jnp.float32,
                               precision=lax.Precision.HIGHEST)
        nrow = lax.dot_general(ones_row, xsq, (((1,), (1,)), ((), ())),
                               preferred_element_type=jnp.float32,
                               precision=lax.Precision.HIGHEST)
        nc_t = ncol[0:H]
        nc_b = ncol[H:N]
        s_tl = nc_t + nrow[:, 0:H] - 2.0 * g_top[:, 0:H]
        s_tr = nc_t + nrow[:, H:N] - 2.0 * g_top[:, H:N]
        s_br = nc_b + nrow[:, H:N] - 2.0 * g_br
        t_sq = jnp.where(ii < jj, s_tl, jnp.where(ii > jj, s_br, 0.0))
        t = jnp.sqrt(jnp.maximum(t_sq, 0.0))
        a = jnp.sqrt(jnp.maximum(s_tr, 0.0))
        return t, a

    t1, a1 = dmat(x1_ref[0])
    t2, a2 = dmat(x2_ref[0])
    dist_ref[0, 0, 0] = t1
    dist_ref[0, 0, 1] = a1
    dist_ref[1, 0, 0] = t2
    dist_ref[1, 0, 1] = a2
    max_ref[0, 0, 0] = jnp.maximum(
        jnp.maximum(jnp.max(t1), jnp.max(a1)),
        jnp.maximum(jnp.max(t2), jnp.max(a2)))


def _hist_body(dists_hbm, maxs_hbm, out_hbm, maxs_v, buf, hist, pub,
               sem0, sem1):
    c = lax.axis_index("c")
    s = lax.axis_index("s")
    pltpu.sync_copy(maxs_hbm, maxs_v)
    lane = lax.iota(jnp.int32, NS)
    i16v = lane * 16
    ones_v = jnp.ones((16,), jnp.float32)
    zero_v = jnp.zeros((16,), jnp.float32)
    sems = (sem0, sem1)

    scale_vec = (BINS * 16.0) / maxs_v[...]

    copies = [pltpu.async_copy(dists_hbm.at[c, 0, s], buf.at[0], sems[0])]

    @plsc.parallel_loop(0, (B * STRIDE) // 64)
    def zero_body(i):
        base = i * 64
        for k in range(4):
            hist[pl.ds(base + k * 16, 16)] = zero_v

    for b in range(B):
        copies[b].wait()
        if b + 1 < B:
            copies.append(pltpu.async_copy(
                dists_hbm.at[c, b + 1, s], buf.at[(b + 1) & 1],
                sems[(b + 1) & 1]))
        scale = scale_vec[b]
        base_v = lane + (b * STRIDE)
        cur = b & 1

        @plsc.parallel_loop(0, CHUNK // 128, unroll=8)
        def bin_body(i):
            base = i * 128
            for k in range(8):
                x = buf[cur, pl.ds(base + k * 16, 16)]
                v = (x * scale).astype(jnp.int32)
                addr = base_v + jnp.bitwise_and(v, -16)
                plsc.addupdate_scatter(hist, [addr], ones_v)

    for b in range(B):
        @plsc.parallel_loop(0, BINS // 16 - 1)
        def red_body(g):
            gbase = b * STRIDE + g * 256
            acc = plsc.load_gather(hist, [i16v + gbase])
            for l in range(1, NS):
                acc = acc + plsc.load_gather(hist, [i16v + (gbase + l)])
            pub[b, pl.ds(g * 16, 16)] = acc

        gbase = b * STRIDE + (BINS // 16 - 1) * 256
        acc = plsc.load_gather(hist, [i16v + gbase])
        for l in range(1, NS):
            acc = acc + plsc.load_gather(hist, [i16v + (gbase + l)])
        ov = hist[pl.ds(b * STRIDE + BINS * NS, 16)]
        acc = acc + jnp.where(lane == NS - 1, jnp.sum(ov), 0.0)
        pub[b, pl.ds(BINS - 16, 16)] = acc

    pltpu.sync_copy(pub, out_hbm.at[c, s])


def _jsd_body(counts_ref, maxs_ref, out_ref):
    cts = counts_ref[...]
    mxv = maxs_ref[...][0:B]
    summed = jnp.sum(cts, axis=1)
    c1 = summed[0]
    c2 = summed[1]
    j = lax.broadcasted_iota(jnp.int32, (B, BINS), 1).astype(jnp.float32)
    w = mxv * ((j + 1.0) * (1.0 / BINS)) - mxv * (j * (1.0 / BINS))
    mw = float(MH) * w
    px = c1 / mw
    qx = c2 / mw
    pm = (px + qx) * 0.5
    lpm = jnp.log(pm + EPS)
    e1 = jnp.sum(px * (jnp.log(px + EPS) - lpm), axis=1, keepdims=True)
    e2 = jnp.sum(qx * (jnp.log(qx + EPS) - lpm), axis=1, keepdims=True)
    out_ref[...] = (e1 + e2) * 0.5


def _make_hist_kernel():
    mesh = plsc.VectorSubcoreMesh(core_axis_name="c", subcore_axis_name="s")
    return pl.kernel(
        _hist_body,
        out_type=jax.ShapeDtypeStruct((NC, NS, B, BINS), jnp.float32),
        mesh=mesh,
        compiler_params=pltpu.CompilerParams(needs_layout_passes=False),
        scratch_types=[
            pltpu.VMEM((16,), jnp.float32),
            pltpu.VMEM((2, CHUNK), jnp.float32),
            pltpu.VMEM((B * STRIDE,), jnp.float32),
            pltpu.VMEM((B, BINS), jnp.float32),
            pltpu.SemaphoreType.DMA,
            pltpu.SemaphoreType.DMA,
        ],
    )


def kernel(data1, data2):
    dists, maxs = pl.pallas_call(
        _dist_body,
        grid=(B,),
        in_specs=[
            pl.BlockSpec((1, N, D), lambda b: (b, 0, 0)),
            pl.BlockSpec((1, N, D), lambda b: (b, 0, 0)),
        ],
        out_specs=[
            pl.BlockSpec((2, 1, 2, H, H), lambda b: (0, b, 0, 0, 0)),
            pl.BlockSpec((1, 1, 1), lambda b: (b, 0, 0),
                         memory_space=pltpu.SMEM),
        ],
        out_shape=[
            jax.ShapeDtypeStruct((2, B, 2, H, H), jnp.float32),
            jax.ShapeDtypeStruct((16, 1, 1), jnp.float32),
        ],
    )(data1, data2)

    dists_r = dists.reshape(2, B, NS, CHUNK)

    counts = _make_hist_kernel()(dists_r, maxs.reshape(16))

    jsd = pl.pallas_call(
        _jsd_body,
        in_specs=[
            pl.BlockSpec((NC, NS, B, BINS), lambda: (0, 0, 0, 0)),
            pl.BlockSpec((16, 1), lambda: (0, 0)),
        ],
        out_specs=pl.BlockSpec((B, 1), lambda: (0, 0)),
        out_shape=jax.ShapeDtypeStruct((B, 1), jnp.float32),
    )(counts, maxs.reshape(16, 1))
    return jsd.reshape(B)
